# baseline (speedup 1.0000x reference)
import functools

import jax
import jax.numpy as jnp
from jax import lax
from jax.experimental import pallas as pl
from jax.experimental.pallas import tpu as pltpu
from jax.experimental.pallas import tpu_sc as plsc

_B, _N, _E = 4, 8192, 768
_M = 64
_TOP_K = 8
_EPS = 1e-06

_TB = 2048

_NC, _NS = 2, 16
_NW = _NC * _NS
_NROW = _B * _N
_RT, _CT = _NROW // 8, _E // 128
_RT_W = _RT // _NW
_CROWS = 8
_NCHUNK = _RT_W // _CROWS


def _router_body(wt_ref, bias_ref, x_ref, logits_ref, idx_ref, w_ref):
    x = x_ref[...]
    logits = jnp.dot(x, wt_ref[...], preferred_element_type=jnp.float32)
    logits_ref[...] = logits

    lt = logits.T
    m = jnp.max(lt, axis=0, keepdims=True)
    ex = jnp.exp(lt - m)
    probs = ex / jnp.sum(ex, axis=0, keepdims=True)
    adj = probs + bias_ref[...].T

    iota = lax.broadcasted_iota(jnp.int32, (_M, _TB), 0).astype(jnp.float32)
    work = adj
    idxs, ws = [], []
    for _ in range(_TOP_K):
        mx = jnp.max(work, axis=0, keepdims=True)
        ik = jnp.min(jnp.where(work == mx, iota, float(_M)), axis=0,
                     keepdims=True)
        hit = iota == ik
        wk = jnp.sum(jnp.where(hit, probs, 0.0), axis=0, keepdims=True)
        work = jnp.where(hit, -jnp.inf, work)
        idxs.append(ik)
        ws.append(wk)

    idx_t = jnp.concatenate(idxs, axis=0)
    w_t = jnp.concatenate(ws, axis=0)
    w_t = w_t / (jnp.sum(w_t, axis=0, keepdims=True) + _EPS)
    idx_ref[...] = idx_t.T.astype(jnp.int32)
    w_ref[...] = w_t.T


def _sc_copy_body(src_ref, dst_ref, buf0, buf1, sem0, sem1):
    wid = lax.axis_index("s") * _NC + lax.axis_index("c")
    base = wid * _RT_W
    pltpu.async_copy(src_ref.at[pl.ds(base, _CROWS)], buf0, sem0)

    def pair(j, _):
        i0 = j * 2
        r0 = base + i0 * _CROWS

        @pl.when(i0 + 1 < _NCHUNK)
        def _():
            pltpu.async_copy(src_ref.at[pl.ds(r0 + _CROWS, _CROWS)], buf1, sem1)

        pltpu.make_async_copy(
            src_ref.at[pl.ds(r0, _CROWS)], buf0, sem0
        ).wait()
        pltpu.sync_copy(buf0, dst_ref.at[pl.ds(r0, _CROWS)])

        @pl.when(i0 + 2 < _NCHUNK)
        def _():
            pltpu.async_copy(
                src_ref.at[pl.ds(r0 + 2 * _CROWS, _CROWS)], buf0, sem0
            )

        @pl.when(i0 + 1 < _NCHUNK)
        def _():
            pltpu.make_async_copy(
                src_ref.at[pl.ds(r0 + _CROWS, _CROWS)], buf1, sem1
            ).wait()
            pltpu.sync_copy(buf1, dst_ref.at[pl.ds(r0 + _CROWS, _CROWS)])
        return _

    lax.fori_loop(0, (_NCHUNK + 1) // 2, pair, None)


@functools.partial(
    pl.kernel,
    out_type=jax.ShapeDtypeStruct((_RT, _CT, 8, 128), jnp.float32),
    mesh=plsc.VectorSubcoreMesh(core_axis_name="c", subcore_axis_name="s"),
    scratch_types=[
        pltpu.VMEM((_CROWS, _CT, 8, 128), jnp.float32),
        pltpu.VMEM((_CROWS, _CT, 8, 128), jnp.float32),
        pltpu.SemaphoreType.DMA,
        pltpu.SemaphoreType.DMA,
    ],
)
def _sc_copy(src_ref, dst_ref, buf0, buf1, sem0, sem1):
    _sc_copy_body(src_ref, dst_ref, buf0, buf1, sem0, sem1)


@jax.jit
def _router(seq2d, wt, bias2d):
    n_tok = seq2d.shape[0]
    grid = (n_tok // _TB,)
    logits, idx, wv = pl.pallas_call(
        _router_body,
        grid=grid,
        in_specs=[
            pl.BlockSpec((_E, _M), lambda i: (0, 0)),
            pl.BlockSpec((1, _M), lambda i: (0, 0)),
            pl.BlockSpec((_TB, _E), lambda i: (i, 0)),
        ],
        out_specs=[
            pl.BlockSpec((_TB, _M), lambda i: (i, 0)),
            pl.BlockSpec((_TB, _TOP_K), lambda i: (i, 0)),
            pl.BlockSpec((_TB, _TOP_K), lambda i: (i, 0)),
        ],
        out_shape=[
            jax.ShapeDtypeStruct((n_tok, _M), jnp.float32),
            jax.ShapeDtypeStruct((n_tok, _TOP_K), jnp.int32),
            jax.ShapeDtypeStruct((n_tok, _TOP_K), jnp.float32),
        ],
    )(wt, bias2d, seq2d)
    seq4 = seq2d.reshape(_RT, 8, _CT, 128).transpose(0, 2, 1, 3)
    out4 = _sc_copy(seq4)
    seq_out = out4.transpose(0, 2, 1, 3).reshape(_NROW, _E)
    return logits, idx, wv, seq_out


def kernel(seq, W, bias):
    b, n, e = seq.shape
    seq2d = seq.reshape(b * n, e)
    wt = W.T
    bias2d = bias.reshape(1, _M)
    logits, idx, wv, seq_out = _router(seq2d, wt, bias2d)
    return (
        logits.reshape(b, n, _M),
        idx.reshape(b, n, _TOP_K),
        seq_out.reshape(b, n, e),
        wv.reshape(b, n, _TOP_K),
    )


# speedup vs baseline: 1.3821x; 1.3821x over previous
import jax
import jax.numpy as jnp
from jax import lax
from jax.experimental import pallas as pl
from jax.experimental.pallas import tpu as pltpu

_B, _N, _E = 4, 8192, 768
_M = 64
_TOP_K = 8
_EPS = 1e-06

_TB = 4096


def _router_body(wt_ref, bias_ref, x_ref, logits_ref, idx_ref, w_ref, seq_out_ref):
    x = x_ref[...]
    seq_out_ref[...] = x
    logits = jnp.dot(x, wt_ref[...], preferred_element_type=jnp.float32)
    logits_ref[...] = logits

    lt = logits.T
    m = jnp.max(lt, axis=0, keepdims=True)
    ex = jnp.exp(lt - m)
    probs = ex / jnp.sum(ex, axis=0, keepdims=True)
    adj = probs + bias_ref[...].T

    iota = lax.broadcasted_iota(jnp.int32, (_M, _TB), 0).astype(jnp.float32)
    work = adj
    idxs, ws = [], []
    for _ in range(_TOP_K):
        mx = jnp.max(work, axis=0, keepdims=True)
        ik = jnp.min(jnp.where(work == mx, iota, float(_M)), axis=0,
                     keepdims=True)
        hit = iota == ik
        wk = jnp.sum(jnp.where(hit, probs, 0.0), axis=0, keepdims=True)
        work = jnp.where(hit, -jnp.inf, work)
        idxs.append(ik)
        ws.append(wk)

    idx_t = jnp.concatenate(idxs, axis=0)
    w_t = jnp.concatenate(ws, axis=0)
    w_t = w_t / (jnp.sum(w_t, axis=0, keepdims=True) + _EPS)
    idx_ref[...] = idx_t.T.astype(jnp.int32)
    w_ref[...] = w_t.T


@jax.jit
def _router(seq2d, wt, bias2d):
    n_tok = seq2d.shape[0]
    grid = (n_tok // _TB,)
    return pl.pallas_call(
        _router_body,
        grid=grid,
        compiler_params=pltpu.CompilerParams(vmem_limit_bytes=117440512),
        in_specs=[
            pl.BlockSpec((_E, _M), lambda i: (0, 0)),
            pl.BlockSpec((1, _M), lambda i: (0, 0)),
            pl.BlockSpec((_TB, _E), lambda i: (i, 0)),
        ],
        out_specs=[
            pl.BlockSpec((_TB, _M), lambda i: (i, 0)),
            pl.BlockSpec((_TB, _TOP_K), lambda i: (i, 0)),
            pl.BlockSpec((_TB, _TOP_K), lambda i: (i, 0)),
            pl.BlockSpec((_TB, _E), lambda i: (i, 0)),
        ],
        out_shape=[
            jax.ShapeDtypeStruct((n_tok, _M), jnp.float32),
            jax.ShapeDtypeStruct((n_tok, _TOP_K), jnp.int32),
            jax.ShapeDtypeStruct((n_tok, _TOP_K), jnp.float32),
            jax.ShapeDtypeStruct((n_tok, _E), jnp.float32),
        ],
    )(wt, bias2d, seq2d)


def kernel(seq, W, bias):
    b, n, e = seq.shape
    seq2d = seq.reshape(b * n, e)
    wt = W.T
    bias2d = bias.reshape(1, _M)
    logits, idx, wv, seq_out = _router(seq2d, wt, bias2d)
    return (
        logits.reshape(b, n, _M),
        idx.reshape(b, n, _TOP_K),
        seq_out.reshape(b, n, e),
        wv.reshape(b, n, _TOP_K),
    )
